# baseline (speedup 1.0000x reference)
import functools

import jax
import jax.numpy as jnp
from jax import lax
from jax.experimental import pallas as pl
from jax.experimental.pallas import tpu as pltpu
from jax.experimental.pallas import tpu_sc as plsc

_B, _L, _D = 1024, 200, 64
_N = _B * _L
_NC, _NS = 2, 16
_NW = _NC * _NS
_ROWS = 128
_PER_W = _N // _NW
_K = _PER_W // _ROWS
_NIDX = 6
_NBUF = 5
_G = _K // _NBUF
_T = 4
_V = 100000
_NSEG = 8
_NFUSED = 16 * _NSEG
_LANES = 16


def _embed_body(cats_hbm, tt_hbm, seg_hbm, big_hbm, small_hbm, out_hbm,
                cats_v, tt_v, seg_v, stage_v, acc_v, sem_g0, sem_add, sem_wb):
    w = lax.axis_index("s") * _NC + lax.axis_index("c")
    tok0 = w * _PER_W
    pltpu.sync_copy(cats_hbm.at[pl.ds(tok0 * _T, _PER_W * _T)], cats_v)
    pltpu.sync_copy(tt_hbm.at[pl.ds(tok0, _PER_W)], tt_v)
    pltpu.sync_copy(seg_hbm.at[pl.ds(tok0, _PER_W)], seg_v)

    lane = lax.iota(jnp.int32, _LANES)

    def build_stage(k, b):
        koff = k * _ROWS
        for g16 in range(_ROWS // _LANES):
            off = koff + g16 * _LANES
            s = pl.ds(g16 * _LANES, _LANES)
            rows = off + lane
            rows4 = rows * _T
            for j in range(_T):
                v = plsc.load_gather(cats_v, [rows4 + j])
                stage_v[b, j, s] = v + j * _V
            ttv = plsc.load_gather(tt_v, [rows])
            sgv = plsc.load_gather(seg_v, [rows])
            stage_v[b, 4, s] = ttv * _NSEG + sgv
            stage_v[b, 5, s] = _NFUSED + lax.rem(tok0 + rows, _L)

    def out_slice(k):
        return out_hbm.at[pl.ds((w * _K + k) * _ROWS, _ROWS)]

    def fire_g0(k, b):
        del k
        pltpu.async_copy(big_hbm.at[stage_v.at[b, 0]], acc_v.at[b],
                         sem_g0.at[b])

    def fire_adds(k, b):
        del k
        pltpu.make_async_copy(
            big_hbm.at[stage_v.at[b, 0]], acc_v.at[b], sem_g0.at[b]).wait()
        for j in range(1, _T):
            pltpu.async_copy(big_hbm.at[stage_v.at[b, j]], acc_v.at[b],
                             sem_add.at[b], add=True)
        for j in range(_T, _NIDX):
            pltpu.async_copy(small_hbm.at[stage_v.at[b, j]], acc_v.at[b],
                             sem_add.at[b], add=True)

    def fire_wb(k, b):
        for _ in range(_NIDX - 1):
            pltpu.make_async_copy(
                big_hbm.at[stage_v.at[b, 1]], acc_v.at[b],
                sem_add.at[b]).wait()
        pltpu.async_copy(acc_v.at[b], out_slice(k), sem_wb.at[b])

    def wait_wb(k, b):
        pltpu.make_async_copy(acc_v.at[b], out_slice(k), sem_wb.at[b]).wait()

    for b in range(_NBUF):
        build_stage(b, b)
        fire_g0(b, b)
    for b in range(_NBUF):
        fire_adds(b, b)

    def outer(g, carry):
        for b in range(_NBUF):
            fire_wb((g - 1) * _NBUF + b, b)
        for b in range(_NBUF):
            wait_wb((g - 1) * _NBUF + b, b)
            build_stage(g * _NBUF + b, b)
            fire_g0(g * _NBUF + b, b)
        for b in range(_NBUF):
            fire_adds(g * _NBUF + b, b)
        return carry

    lax.fori_loop(1, _G, outer, 0)

    for b in range(_NBUF):
        fire_wb((_G - 1) * _NBUF + b, b)
    for b in range(_NBUF):
        wait_wb((_G - 1) * _NBUF + b, b)


_embed = functools.partial(
    pl.kernel,
    out_type=jax.ShapeDtypeStruct((_N, _D), jnp.float32),
    mesh=plsc.VectorSubcoreMesh(core_axis_name="c", subcore_axis_name="s"),
    scratch_types=[
        pltpu.VMEM((_PER_W * _T,), jnp.int32),
        pltpu.VMEM((_PER_W,), jnp.int32),
        pltpu.VMEM((_PER_W,), jnp.int32),
        pltpu.VMEM((_NBUF, _NIDX, _ROWS), jnp.int32),
        pltpu.VMEM((_NBUF, _ROWS, _D), jnp.float32),
        pltpu.SemaphoreType.DMA((_NBUF,)),
        pltpu.SemaphoreType.DMA((_NBUF,)),
        pltpu.SemaphoreType.DMA((_NBUF,)),
    ],
    compiler_params=pltpu.CompilerParams(use_tc_tiling_on_sc=False,
                                         needs_layout_passes=False),
)(_embed_body)


def kernel(token_types, segments, semantic_embeds, categories,
           token_type_table, segment_table, cat_tables, pe):
    del semantic_embeds
    big = cat_tables.reshape(_T * _V, _D)
    fused_small = (token_type_table[:, None, :]
                   + segment_table[None, :, :]).reshape(-1, _D)
    small = jnp.concatenate([fused_small, pe[0]], axis=0)

    cats_flat = categories.astype(jnp.int32).reshape(_N * _T)
    tt_flat = token_types.astype(jnp.int32).reshape(_N)
    seg_flat = segments.astype(jnp.int32).reshape(_N)

    out = _embed(cats_flat, tt_flat, seg_flat, big, small)
    return out.reshape(_B, _L, _D)
